# baseline (speedup 1.0000x reference)
import jax
import jax.numpy as jnp
from jax.experimental import pallas as pl
from jax.experimental.pallas import tpu as pltpu

N = 10000
D = 128
BM = 1024
BKA = 2048
BKB = 4096
NI = (N + BM - 1) // BM
NKA = (N + BKA - 1) // BKA
NKB = (N + BKB - 1) // BKB
NPAD = NI * BM


def _body_a(A_ref, At_ref, xs_ref, xt_ref, y_ref, a16_ref, acc_ref):
    m = pl.program_id(0)
    i = pl.program_id(1)
    k = pl.program_id(2)

    def half(a_ref, x_ref):
        a16 = a_ref[...].astype(jnp.bfloat16)
        a16_ref[0] = a16
        rhs = x_ref[pl.ds(k * BKA, BKA), :].astype(jnp.bfloat16)

        @pl.when(k == 0)
        def _():
            acc_ref[...] = jnp.dot(a16, rhs, preferred_element_type=jnp.float32)

        @pl.when(jnp.logical_and(k > 0, k < NKA - 1))
        def _():
            acc_ref[...] += jnp.dot(a16, rhs, preferred_element_type=jnp.float32)

        @pl.when(k == NKA - 1)
        def _():
            colmask = jax.lax.broadcasted_iota(
                jnp.int32, (BM, BKA), 1) < (N - k * BKA)
            am = jnp.where(colmask, a16, jnp.bfloat16(0))
            acc = acc_ref[...] + jnp.dot(am, rhs,
                                         preferred_element_type=jnp.float32)
            rowmask = jax.lax.broadcasted_iota(
                jnp.int32, (BM, D), 0) < (N - i * BM)
            y_ref[0] = jnp.where(rowmask, acc, 0.0)

    @pl.when(m == 0)
    def _():
        half(A_ref, xs_ref)

    @pl.when(m == 1)
    def _():
        half(At_ref, xt_ref)


def _body_b(a16_ref, xs_ref, xt_ref, y_ref, ws_ref, wt_ref, o_ref, acc_ref):
    m = pl.program_id(0)
    i = pl.program_id(1)
    k = pl.program_id(2)

    def half(x_ref, w_ref):
        lhs = a16_ref[0]
        rhs = (w_ref[1, 0] * x_ref[pl.ds(k * BKB, BKB), :]
               + w_ref[2, 0] * y_ref[0, pl.ds(k * BKB, BKB), :]
               ).astype(jnp.bfloat16)

        @pl.when(k == 0)
        def _():
            acc_ref[...] = jnp.dot(lhs, rhs, preferred_element_type=jnp.float32)

        @pl.when(jnp.logical_and(k > 0, k < NKB - 1))
        def _():
            acc_ref[...] += jnp.dot(lhs, rhs, preferred_element_type=jnp.float32)

        @pl.when(k == NKB - 1)
        def _():
            colmask = jax.lax.broadcasted_iota(
                jnp.int32, (BM, BKB), 1) < (N - k * BKB)
            lm = jnp.where(colmask, lhs, jnp.bfloat16(0))
            acc = acc_ref[...] + jnp.dot(lm, rhs,
                                         preferred_element_type=jnp.float32)
            o_ref[...] = w_ref[0, 0] * x_ref[pl.ds(i * BM, BM), :] + acc

    @pl.when(m == 0)
    def _():
        half(xs_ref, ws_ref)

    @pl.when(m == 1)
    def _():
        half(xt_ref, wt_ref)


def _hop1(xs_pad, xt_pad, A, At, interpret=False):
    return pl.pallas_call(
        _body_a,
        grid=(2, NI, NKA),
        in_specs=[
            pl.BlockSpec((BM, BKA),
                         lambda m, i, k: (jnp.where(m == 0, i, NI - 1),
                                          jnp.where(m == 0, k, NKA - 1))),
            pl.BlockSpec((BM, BKA),
                         lambda m, i, k: (jnp.where(m == 1, i, NI - 1),
                                          jnp.where(m == 1, k, NKA - 1))),
            pl.BlockSpec((NPAD, D), lambda m, i, k: (0, 0)),
            pl.BlockSpec((NPAD, D), lambda m, i, k: (0, 0)),
        ],
        out_specs=[
            pl.BlockSpec((1, BM, D), lambda m, i, k: (m, i, 0)),
            pl.BlockSpec((1, BM, BKA), lambda m, i, k: (m, i, k)),
        ],
        out_shape=[
            jax.ShapeDtypeStruct((2, NPAD, D), jnp.float32),
            jax.ShapeDtypeStruct((2, N, N), jnp.bfloat16),
        ],
        scratch_shapes=[pltpu.VMEM((BM, D), jnp.float32)],
        compiler_params=pltpu.CompilerParams(
            dimension_semantics=("arbitrary",) * 3),
        interpret=interpret,
    )(A, At, xs_pad, xt_pad)


def _hop2(a16, xs_pad, xt_pad, y, w_s, w_t, interpret=False):
    return pl.pallas_call(
        _body_b,
        grid=(2, NI, NKB),
        in_specs=[
            pl.BlockSpec((1, BM, BKB), lambda m, i, k: (m, i, k)),
            pl.BlockSpec((NPAD, D), lambda m, i, k: (0, 0)),
            pl.BlockSpec((NPAD, D), lambda m, i, k: (0, 0)),
            pl.BlockSpec((1, NPAD, D), lambda m, i, k: (m, 0, 0)),
            pl.BlockSpec(memory_space=pltpu.SMEM),
            pl.BlockSpec(memory_space=pltpu.SMEM),
        ],
        out_specs=pl.BlockSpec((BM, D), lambda m, i, k: (i, m)),
        out_shape=jax.ShapeDtypeStruct((N, 2 * D), jnp.float32),
        scratch_shapes=[pltpu.VMEM((BM, D), jnp.float32)],
        compiler_params=pltpu.CompilerParams(
            dimension_semantics=("arbitrary",) * 3),
        interpret=interpret,
    )(a16, xs_pad, xt_pad, y, w_s, w_t)


def kernel(x_s, x_t, A, At, w_s, w_t):
    xs_pad = jnp.pad(x_s, ((0, NPAD - N), (0, 0)))
    xt_pad = jnp.pad(x_t, ((0, NPAD - N), (0, 0)))
    y, a16 = _hop1(xs_pad, xt_pad, A, At)
    return _hop2(a16, xs_pad, xt_pad, y, w_s, w_t)


# speedup vs baseline: 1.2574x; 1.2574x over previous
import jax
import jax.numpy as jnp
from jax.experimental import pallas as pl
from jax.experimental.pallas import tpu as pltpu

N = 10000
D = 128
BM = 1024
BK = 2048
NI = (N + BM - 1) // BM
NK = (N + BK - 1) // BK
NPAD = NI * BM


def _body(A_ref, At_ref, xs_ref, xt_ref, ws_ref, wt_ref,
          o_ref, acc_ref, ys_ref, yt_ref, xps_ref, xpt_ref):
    p = pl.program_id(0)
    m = pl.program_id(1)
    i = pl.program_id(2)
    k = pl.program_id(3)

    @pl.when(jnp.logical_and(jnp.logical_and(p == 0, m == 0),
                             jnp.logical_and(i == 0, k == 0)))
    def _():
        xps_ref[0:N, :] = xs_ref[...]
        xps_ref[N:NPAD, :] = jnp.zeros((NPAD - N, D), jnp.float32)
        xpt_ref[0:N, :] = xt_ref[...]
        xpt_ref[N:NPAD, :] = jnp.zeros((NPAD - N, D), jnp.float32)

    def masked_a(a_ref):
        rem_k = N - k * BK
        colmask = jax.lax.broadcasted_iota(jnp.int32, (BM, BK), 1) < rem_k
        return jnp.where(colmask, a_ref[...], 0.0)

    def steps(a_ref, rhs_fn, epilogue):
        @pl.when(k == 0)
        def _():
            acc_ref[...] = jnp.dot(a_ref[...], rhs_fn(),
                                   preferred_element_type=jnp.float32)

        @pl.when(jnp.logical_and(k > 0, k < NK - 1))
        def _():
            acc_ref[...] += jnp.dot(a_ref[...], rhs_fn(),
                                    preferred_element_type=jnp.float32)

        @pl.when(k == NK - 1)
        def _():
            acc = acc_ref[...] + jnp.dot(masked_a(a_ref), rhs_fn(),
                                         preferred_element_type=jnp.float32)
            epilogue(acc)

    def hop1(a_ref, x_ref, y_ref):
        def rhs_fn():
            return x_ref[pl.ds(k * BK, BK), :]

        def epilogue(acc):
            rem_m = N - i * BM
            rowmask = jax.lax.broadcasted_iota(jnp.int32, (BM, D), 0) < rem_m
            y_ref[pl.ds(i * BM, BM), :] = jnp.where(rowmask, acc, 0.0)

        steps(a_ref, rhs_fn, epilogue)

    def hop2(a_ref, x_ref, y_ref, w_ref):
        def rhs_fn():
            return (w_ref[1, 0] * x_ref[pl.ds(k * BK, BK), :]
                    + w_ref[2, 0] * y_ref[pl.ds(k * BK, BK), :])

        def epilogue(acc):
            o_ref[...] = w_ref[0, 0] * x_ref[pl.ds(i * BM, BM), :] + acc

        steps(a_ref, rhs_fn, epilogue)

    @pl.when(jnp.logical_and(p == 0, m == 0))
    def _():
        hop1(A_ref, xps_ref, ys_ref)

    @pl.when(jnp.logical_and(p == 0, m == 1))
    def _():
        hop1(At_ref, xpt_ref, yt_ref)

    @pl.when(jnp.logical_and(p == 1, m == 0))
    def _():
        hop2(A_ref, xps_ref, ys_ref, ws_ref)

    @pl.when(jnp.logical_and(p == 1, m == 1))
    def _():
        hop2(At_ref, xpt_ref, yt_ref, wt_ref)


def _feat(x_s, x_t, A, At, w_s, w_t, interpret=False):
    return pl.pallas_call(
        _body,
        grid=(2, 2, NI, NK),
        in_specs=[
            pl.BlockSpec((BM, BK),
                         lambda p, m, i, k: (jnp.where(m == 0, i, NI - 1),
                                             jnp.where(m == 0, k, NK - 1))),
            pl.BlockSpec((BM, BK),
                         lambda p, m, i, k: (jnp.where(m == 1, i, NI - 1),
                                             jnp.where(m == 1, k, NK - 1))),
            pl.BlockSpec((N, D), lambda p, m, i, k: (0, 0)),
            pl.BlockSpec((N, D), lambda p, m, i, k: (0, 0)),
            pl.BlockSpec(memory_space=pltpu.SMEM),
            pl.BlockSpec(memory_space=pltpu.SMEM),
        ],
        out_specs=pl.BlockSpec((BM, D),
                               lambda p, m, i, k: (jnp.where(p == 0, 0, i), m)),
        out_shape=jax.ShapeDtypeStruct((N, 2 * D), jnp.float32),
        scratch_shapes=[
            pltpu.VMEM((BM, D), jnp.float32),
            pltpu.VMEM((NPAD, D), jnp.float32),
            pltpu.VMEM((NPAD, D), jnp.float32),
            pltpu.VMEM((NPAD, D), jnp.float32),
            pltpu.VMEM((NPAD, D), jnp.float32),
        ],
        compiler_params=pltpu.CompilerParams(
            dimension_semantics=("arbitrary",) * 4,
            vmem_limit_bytes=100 * 1024 * 1024),
        interpret=interpret,
    )(A, At, x_s, x_t, w_s, w_t)


def kernel(x_s, x_t, A, At, w_s, w_t):
    return _feat(x_s, x_t, A, At, w_s, w_t)
